# baseline (speedup 1.0000x reference)
import jax
import jax.numpy as jnp
from jax import lax
from jax.experimental import pallas as pl
from jax.experimental.pallas import tpu as pltpu
from jax.experimental.pallas import tpu_sc as plsc

_N = 10000
_E = 320000
_NC, _NS = 2, 16
_NW = _NC * _NS
_B = 128
_CPW = 80
_NBUF = 2
_EPAD = _NW * _CPW * _B
_SPT = 632
_LAST = _N - (_NS - 1) * _SPT
_GROW = 128
_D1 = 64
_D2 = 48

_mesh = plsc.VectorSubcoreMesh(core_axis_name="c", subcore_axis_name="s")


def _iota16():
    return lax.iota(jnp.int32, 16)


def _fori(hi, body):
    return lax.fori_loop(jnp.int32(0), jnp.int32(hi),
                         lambda i, carry: (body(i), carry)[1], jnp.int32(0))


def _zero_acc_stripe(s, msg0, acc):
    d = msg0.shape[-1]
    z16 = jnp.zeros((16,), jnp.float32)

    def _zrow(e):
        for q in range(d // 16):
            msg0[e, pl.ds(16 * q, 16)] = z16
    _fori(8, _zrow)

    base_r = jnp.int32(s) * _SPT
    nz8 = jnp.where(s == _NS - 1, jnp.int32(_LAST // 8), jnp.int32(_SPT // 8))

    def _zacc(r):
        pltpu.sync_copy(msg0.at[pl.ds(0, 8)], acc.at[pl.ds(base_r + 8 * r, 8)])
    _fori(nz8, _zacc)


def _stage_table(s, hbm_ref, sp_ref):
    @pl.when(s < _NS - 1)
    def _():
        base = jnp.int32(s) * _SPT
        pltpu.sync_copy(hbm_ref.at[pl.ds(base, _SPT)],
                        sp_ref.at[pl.ds(base, _SPT)])

    @pl.when(s == _NS - 1)
    def _():
        pltpu.sync_copy(hbm_ref.at[pl.ds((_NS - 1) * _SPT, _LAST)],
                        sp_ref.at[pl.ds((_NS - 1) * _SPT, _LAST)])


def _copy_out_stripe(c, s, acc, out_ref):
    @pl.when(s < _NS - 1)
    def _():
        base = jnp.int32(s) * _SPT
        pltpu.sync_copy(acc.at[pl.ds(base, _SPT)],
                        out_ref.at[c, pl.ds(base, _SPT)])

    @pl.when(s == _NS - 1)
    def _():
        pltpu.sync_copy(acc.at[pl.ds((_NS - 1) * _SPT, _LAST)],
                        out_ref.at[c, pl.ds((_NS - 1) * _SPT, _LAST)])


def _sc_l1_body(src_ref, dst_ref, asd_ref, xwe_ref, out_ref, outd_ref,
                sidx2, didx2, asg, adg, xwg, wbuf, msg, acc, accd,
                gsem0, gsem1, ssem0, ssem1):
    c = lax.axis_index("c")
    s = lax.axis_index("s")
    wid = jnp.int32(s) * _NC + jnp.int32(c)
    it = _iota16()
    shift = (it & 7) + 8
    pats = [2 * q + (it >> 3) for q in range(4)]
    gsems = (gsem0, gsem1)
    ssems = (ssem0, ssem1)

    pltpu.sync_copy(src_ref.at[pl.ds(wid * _CPW, _CPW)], sidx2)
    pltpu.sync_copy(dst_ref.at[pl.ds(wid * _CPW, _CPW)], didx2)
    _zero_acc_stripe(s, msg.at[jnp.int32(0)], acc)
    _zero_acc_stripe(s, wbuf.at[jnp.int32(0)], accd)
    plsc.subcore_barrier()

    def _gather_cps(k, b):
        return (
            pltpu.make_async_copy(asd_ref.at[sidx2.at[k]], asg.at[jnp.int32(b)], gsems[b]),
            pltpu.make_async_copy(asd_ref.at[didx2.at[k]], adg.at[jnp.int32(b)], gsems[b]),
            pltpu.make_async_copy(xwe_ref.at[sidx2.at[k]], xwg.at[jnp.int32(b)], gsems[b]),
        )

    def _issue(k, b):
        for cp in _gather_cps(k, b):
            cp.start()

    def _wait_gathers(k, b):
        for cp in _gather_cps(k, b):
            cp.wait()

    def _scatter_cps(k, b):
        return (
            pltpu.make_async_copy(msg.at[jnp.int32(b)], acc.at[didx2.at[k]], ssems[b]),
            pltpu.make_async_copy(wbuf.at[jnp.int32(b)], accd.at[didx2.at[k]], ssems[b]),
        )

    def _compute(b):
        bi = jnp.int32(b)
        asgb, adgb, xwgb, wbufb, msgb = (
            asg.at[bi], adg.at[bi], xwg.at[bi], wbuf.at[bi], msg.at[bi])

        @plsc.parallel_loop(jnp.int32(0), jnp.int32(_B), step=jnp.int32(1), unroll=4)
        def _edge_w(e):
            esp = jnp.full((16,), e, jnp.int32)
            vas = asgb[e, :]
            vad = plsc.load_gather(adgb, [esp, shift])
            lg = vas + vad
            lg = jnp.where(lg > 0, lg, 0.2 * lg)
            wbufb[e, :] = jnp.exp(lg)

        @plsc.parallel_loop(jnp.int32(0), jnp.int32(_B), step=jnp.int32(1), unroll=2)
        def _edge_m(e):
            esp = jnp.full((16,), e, jnp.int32)
            for q2 in range(_D1 // 32):
                v32 = xwgb[e, pl.ds(32 * q2, 32)]
                va, vb = plsc.unpack(v32, format=plsc.PackFormat.INTERLEAVED)
                for j, vv in enumerate((va, vb)):
                    q = 2 * q2 + j
                    wq = plsc.load_gather(wbufb, [esp, pats[q]])
                    msgb[e, pl.ds(16 * q, 16)] = wq * vv

    for b in range(_NBUF):
        _issue(jnp.int32(b), b)

    def _outer(t):
        for b in range(_NBUF):
            k = _NBUF * t + b
            _wait_gathers(k, b)

            @pl.when(t > 0)
            def _():
                for cp in _scatter_cps(k - _NBUF, b):
                    cp.wait()

            _compute(b)
            for cp in _scatter_cps(k, b):
                cp.start(add=True)

            @pl.when(t < _CPW // _NBUF - 1)
            def _():
                _issue(k + _NBUF, b)
    _fori(_CPW // _NBUF, _outer)
    for b in range(_NBUF):
        for cp in _scatter_cps(jnp.int32(_CPW - _NBUF + b), b):
            cp.wait()

    plsc.subcore_barrier()
    _copy_out_stripe(c, s, acc, out_ref)
    _copy_out_stripe(c, s, accd, outd_ref)


_sc_l1 = pl.kernel(
    _sc_l1_body,
    out_type=[jax.ShapeDtypeStruct((_NC, _N, _D1), jnp.float32),
              jax.ShapeDtypeStruct((_NC, _N, 16), jnp.float32)],
    mesh=_mesh,
    scratch_types=[
        pltpu.VMEM((_CPW, _B), jnp.int32),
        pltpu.VMEM((_CPW, _B), jnp.int32),
        pltpu.VMEM((_NBUF, _B, 16), jnp.float32),
        pltpu.VMEM((_NBUF, _B, 16), jnp.float32),
        pltpu.VMEM((_NBUF, _B, _D1), jnp.bfloat16),
        pltpu.VMEM((_NBUF, _B, 16), jnp.float32),
        pltpu.VMEM((_NBUF, _B, _D1), jnp.float32),
        pltpu.VMEM_SHARED((_N + _GROW, _D1), jnp.float32),
        pltpu.VMEM_SHARED((_N + _GROW, 16), jnp.float32),
        pltpu.SemaphoreType.DMA,
        pltpu.SemaphoreType.DMA,
        pltpu.SemaphoreType.DMA,
        pltpu.SemaphoreType.DMA,
    ],
    compiler_params=pltpu.CompilerParams(needs_layout_passes=False,
                                         use_tc_tiling_on_sc=False),
)


def _sc_l2_body(src_ref, dst_ref, a2s_ref, a2d_ref, hwe_ref, out_ref,
                sidx2, didx2, a2sv, a2dv, hwg, wbuf, msg, acc, hwe_s,
                gsem0, gsem1, ssem0, ssem1):
    c = lax.axis_index("c")
    s = lax.axis_index("s")
    wid = jnp.int32(s) * _NC + jnp.int32(c)
    gsems = (gsem0, gsem1)
    ssems = (ssem0, ssem1)

    pltpu.sync_copy(src_ref.at[pl.ds(wid * _CPW, _CPW)], sidx2)
    pltpu.sync_copy(dst_ref.at[pl.ds(wid * _CPW, _CPW)], didx2)
    pltpu.sync_copy(a2s_ref, a2sv)
    pltpu.sync_copy(a2d_ref, a2dv)
    _stage_table(s, hwe_ref, hwe_s)
    _zero_acc_stripe(s, msg.at[jnp.int32(0)], acc)
    plsc.subcore_barrier()

    def _gather_cp(k, b):
        return pltpu.make_async_copy(hwe_s.at[sidx2.at[k]],
                                     hwg.at[jnp.int32(b)], gsems[b])

    def _scatter_cp(k, b):
        return pltpu.make_async_copy(msg.at[jnp.int32(b)], acc.at[didx2.at[k]], ssems[b])

    def _compute(k, b):
        bi = jnp.int32(b)
        hwgb, wbufb, msgb = hwg.at[bi], wbuf.at[bi], msg.at[bi]

        @plsc.parallel_loop(jnp.int32(0), jnp.int32(_B // 16), step=jnp.int32(1), unroll=2)
        def _grp(gi):
            sv = sidx2[k, pl.ds(16 * gi, 16)]
            dv = didx2[k, pl.ds(16 * gi, 16)]
            as16 = plsc.load_gather(a2sv, [sv])
            ad16 = plsc.load_gather(a2dv, [dv])
            lg = as16 + ad16
            lg = jnp.where(lg > 0, lg, 0.2 * lg)
            wbufb[pl.ds(16 * gi, 16)] = jnp.exp(lg)

        @plsc.parallel_loop(jnp.int32(0), jnp.int32(_B), step=jnp.int32(1), unroll=4)
        def _edge(e):
            esp = jnp.full((16,), e, jnp.int32)
            spl = plsc.load_gather(wbufb, [esp])
            for q in range(_D2 // 16):
                msgb[e, pl.ds(16 * q, 16)] = spl * hwgb[e, pl.ds(16 * q, 16)]

    for b in range(_NBUF):
        _gather_cp(jnp.int32(b), b).start()

    def _outer(t):
        for b in range(_NBUF):
            k = _NBUF * t + b
            _gather_cp(k, b).wait()

            @pl.when(t > 0)
            def _():
                _scatter_cp(k - _NBUF, b).wait()

            _compute(k, b)
            _scatter_cp(k, b).start(add=True)

            @pl.when(t < _CPW // _NBUF - 1)
            def _():
                _gather_cp(k + _NBUF, b).start()
    _fori(_CPW // _NBUF, _outer)
    for b in range(_NBUF):
        _scatter_cp(jnp.int32(_CPW - _NBUF + b), b).wait()

    plsc.subcore_barrier()
    _copy_out_stripe(c, s, acc, out_ref)


_sc_l2 = pl.kernel(
    _sc_l2_body,
    out_type=jax.ShapeDtypeStruct((_NC, _N, _D2), jnp.float32),
    mesh=_mesh,
    scratch_types=[
        pltpu.VMEM((_CPW, _B), jnp.int32),
        pltpu.VMEM((_CPW, _B), jnp.int32),
        pltpu.VMEM((_N,), jnp.float32),
        pltpu.VMEM((_N,), jnp.float32),
        pltpu.VMEM((_NBUF, _B, _D2), jnp.float32),
        pltpu.VMEM((_NBUF, _B), jnp.float32),
        pltpu.VMEM((_NBUF, _B, _D2), jnp.float32),
        pltpu.VMEM_SHARED((_N + _GROW, _D2), jnp.float32),
        pltpu.VMEM_SHARED((_N, _D2), jnp.float32),
        pltpu.SemaphoreType.DMA,
        pltpu.SemaphoreType.DMA,
        pltpu.SemaphoreType.DMA,
        pltpu.SemaphoreType.DMA,
    ],
    compiler_params=pltpu.CompilerParams(needs_layout_passes=False,
                                         use_tc_tiling_on_sc=False),
)


def _tc1_body(x_ref, w1_ref, as_ref, ad_ref, pm_ref, xwe_ref, asd_ref):
    xw = jnp.dot(x_ref[...], w1_ref[...], preferred_element_type=jnp.float32)
    a_s = jnp.dot(xw, as_ref[...], preferred_element_type=jnp.float32)
    a_d = jnp.dot(xw, ad_ref[...], preferred_element_type=jnp.float32)
    xwp = jnp.dot(xw, pm_ref[...], preferred_element_type=jnp.float32)
    xwe_ref[...] = xwp.astype(jnp.bfloat16)
    asd_ref[...] = jnp.concatenate([a_s, a_d], axis=1)


_tc1 = pl.pallas_call(
    _tc1_body,
    out_shape=[
        jax.ShapeDtypeStruct((_N, _D1), jnp.bfloat16),
        jax.ShapeDtypeStruct((_N, 16), jnp.float32),
    ],
)


def _tc2_body(acc_ref, accd_ref, b1_ref, w2_ref, a2s_ref, a2d_ref, r_ref,
              hwe_ref, a2_ref):
    num = acc_ref[0] + acc_ref[1]
    den8 = accd_ref[0][:, 0:8] + accd_ref[1][:, 0:8]
    den64 = jnp.dot(den8, r_ref[...], preferred_element_type=jnp.float32)
    h = num / jnp.maximum(den64, 1e-30) + b1_ref[...]
    h = jnp.where(h > 0, h, jnp.exp(jnp.minimum(h, 0.0)) - 1.0)
    hw = jnp.dot(h, w2_ref[...], preferred_element_type=jnp.float32)
    a2s = jnp.dot(hw, a2s_ref[...], preferred_element_type=jnp.float32)
    a2d = jnp.dot(hw, a2d_ref[...], preferred_element_type=jnp.float32)
    ones = jnp.ones((_N, 1), jnp.float32)
    zeros = jnp.zeros((_N, 7), jnp.float32)
    hwe_ref[...] = jnp.concatenate([hw, ones, zeros], axis=1)
    a2_ref[...] = jnp.concatenate([a2s, a2d], axis=1)


_tc2 = pl.pallas_call(
    _tc2_body,
    out_shape=[
        jax.ShapeDtypeStruct((_N, _D2), jnp.float32),
        jax.ShapeDtypeStruct((_N, 2), jnp.float32),
    ],
)


def _tc3_body(acc_ref, b2_ref, out_ref):
    num = acc_ref[0] + acc_ref[1]
    den = jnp.maximum(num[:, 40:41], 1e-30)
    lg = num[:, 0:40] / den + b2_ref[...]
    m = jnp.max(lg, axis=1, keepdims=True)
    ls = lg - m
    out_ref[...] = ls - jnp.log(jnp.sum(jnp.exp(ls), axis=1, keepdims=True))


_tc3 = pl.pallas_call(
    _tc3_body,
    out_shape=jax.ShapeDtypeStruct((_N, 40), jnp.float32),
)


def kernel(x, edge_index, y, W1, att_src1, att_dst1, b1, W2, att_src2,
           att_dst2, b2):
    del y
    f32 = jnp.float32
    x = x.astype(f32)
    src = edge_index[0].astype(jnp.int32)
    dst = edge_index[1].astype(jnp.int32)
    npad = _EPAD - _E
    src_p = jnp.pad(src, (0, npad)).reshape(_NW * _CPW, _B)
    dst_p = jnp.pad(dst, (0, npad),
                    constant_values=_N).reshape(_NW * _CPW, _B)

    hh = jnp.repeat(jnp.arange(8), 8)
    hmask = (hh[:, None] == jnp.arange(8)[None, :])
    As = jnp.where(hmask, att_src1.reshape(64).astype(f32)[:, None], 0.0)
    Ad = jnp.where(hmask, att_dst1.reshape(64).astype(f32)[:, None], 0.0)
    R = (jnp.arange(8)[:, None] == hh[None, :]).astype(f32)
    perm = []
    for half in range(2):
        for j in range(16):
            perm += [32 * half + j, 32 * half + 16 + j]
    Pm = (jnp.arange(64)[:, None] == jnp.array(perm)[None, :]).astype(f32)

    xwe, asd = _tc1(x, W1.astype(f32), As, Ad, Pm)
    acc1, acc1d = _sc_l1(src_p, dst_p, asd, xwe)
    hwe, a2 = _tc2(acc1, acc1d, b1.astype(f32).reshape(1, 64), W2.astype(f32),
                   att_src2.astype(f32).reshape(40, 1),
                   att_dst2.astype(f32).reshape(40, 1), R)
    acc2 = _sc_l2(src_p, dst_p, a2[:, 0], a2[:, 1], hwe)
    return _tc3(acc2, b2.astype(f32).reshape(1, 40))


# speedup vs baseline: 1.0026x; 1.0026x over previous
import jax
import jax.numpy as jnp
from jax import lax
from jax.experimental import pallas as pl
from jax.experimental.pallas import tpu as pltpu
from jax.experimental.pallas import tpu_sc as plsc

_N = 10000
_E = 320000
_NC, _NS = 2, 16
_NW = _NC * _NS
_B = 128
_CPW = 80
_NBUF = 2
_EPAD = _NW * _CPW * _B
_SPT = 632
_LAST = _N - (_NS - 1) * _SPT
_GROW = 128
_D1 = 64
_D2 = 48

_mesh = plsc.VectorSubcoreMesh(core_axis_name="c", subcore_axis_name="s")


def _iota16():
    return lax.iota(jnp.int32, 16)


def _fori(hi, body):
    return lax.fori_loop(jnp.int32(0), jnp.int32(hi),
                         lambda i, carry: (body(i), carry)[1], jnp.int32(0))


def _zero_acc_stripe(s, msg0, acc):
    d = msg0.shape[-1]
    z16 = jnp.zeros((16,), jnp.float32)

    def _zrow(e):
        for q in range(d // 16):
            msg0[e, pl.ds(16 * q, 16)] = z16
    _fori(8, _zrow)

    base_r = jnp.int32(s) * _SPT
    nz8 = jnp.where(s == _NS - 1, jnp.int32(_LAST // 8), jnp.int32(_SPT // 8))

    def _zacc(r):
        pltpu.sync_copy(msg0.at[pl.ds(0, 8)], acc.at[pl.ds(base_r + 8 * r, 8)])
    _fori(nz8, _zacc)


def _stage_table(s, hbm_ref, sp_ref):
    @pl.when(s < _NS - 1)
    def _():
        base = jnp.int32(s) * _SPT
        pltpu.sync_copy(hbm_ref.at[pl.ds(base, _SPT)],
                        sp_ref.at[pl.ds(base, _SPT)])

    @pl.when(s == _NS - 1)
    def _():
        pltpu.sync_copy(hbm_ref.at[pl.ds((_NS - 1) * _SPT, _LAST)],
                        sp_ref.at[pl.ds((_NS - 1) * _SPT, _LAST)])


def _copy_out_stripe(c, s, acc, out_ref):
    @pl.when(s < _NS - 1)
    def _():
        base = jnp.int32(s) * _SPT
        pltpu.sync_copy(acc.at[pl.ds(base, _SPT)],
                        out_ref.at[c, pl.ds(base, _SPT)])

    @pl.when(s == _NS - 1)
    def _():
        pltpu.sync_copy(acc.at[pl.ds((_NS - 1) * _SPT, _LAST)],
                        out_ref.at[c, pl.ds((_NS - 1) * _SPT, _LAST)])


def _sc_l1_body(src_ref, dst_ref, asd_ref, xwe_ref, out_ref, outd_ref,
                sidx2, didx2, asg, adg, xwg, wbuf, msg, acc, accd,
                gsem0, gsem1, ssem0, ssem1):
    c = lax.axis_index("c")
    s = lax.axis_index("s")
    wid = jnp.int32(s) * _NC + jnp.int32(c)
    it = _iota16()
    shift = (it & 7) + 8
    pats = [2 * q + (it >> 3) for q in range(4)]
    gsems = (gsem0, gsem1)
    ssems = (ssem0, ssem1)

    pltpu.sync_copy(src_ref.at[pl.ds(wid * _CPW, _CPW)], sidx2)
    pltpu.sync_copy(dst_ref.at[pl.ds(wid * _CPW, _CPW)], didx2)
    _zero_acc_stripe(s, msg.at[jnp.int32(0)], acc)
    _zero_acc_stripe(s, wbuf.at[jnp.int32(0)], accd)
    plsc.subcore_barrier()

    def _gather_cps(k, b):
        return (
            pltpu.make_async_copy(asd_ref.at[sidx2.at[k]], asg.at[jnp.int32(b)], gsems[b]),
            pltpu.make_async_copy(asd_ref.at[didx2.at[k]], adg.at[jnp.int32(b)], gsems[b]),
            pltpu.make_async_copy(xwe_ref.at[sidx2.at[k]], xwg.at[jnp.int32(b)], gsems[b]),
        )

    def _issue(k, b):
        for cp in _gather_cps(k, b):
            cp.start()

    def _wait_gathers(k, b):
        for cp in _gather_cps(k, b):
            cp.wait()

    def _scatter_cps(k, b):
        return (
            pltpu.make_async_copy(msg.at[jnp.int32(b)], acc.at[didx2.at[k]], ssems[b]),
            pltpu.make_async_copy(wbuf.at[jnp.int32(b)], accd.at[didx2.at[k]], ssems[b]),
        )

    def _compute(b):
        bi = jnp.int32(b)
        asgb, adgb, xwgb, wbufb, msgb = (
            asg.at[bi], adg.at[bi], xwg.at[bi], wbuf.at[bi], msg.at[bi])

        @plsc.parallel_loop(jnp.int32(0), jnp.int32(_B), step=jnp.int32(1), unroll=2)
        def _edge(e):
            esp = jnp.full((16,), e, jnp.int32)
            vas = asgb[e, :]
            vad = plsc.load_gather(adgb, [esp, shift])
            lg = vas + vad
            lg = jnp.where(lg > 0, lg, 0.2 * lg)
            w = jnp.exp(lg)
            wbufb[e, :] = w
            for q2 in range(_D1 // 32):
                v32 = xwgb[e, pl.ds(32 * q2, 32)]
                va, vb = plsc.unpack(v32, format=plsc.PackFormat.INTERLEAVED)
                for j, vv in enumerate((va, vb)):
                    q = 2 * q2 + j
                    wq = w[pats[q]]
                    msgb[e, pl.ds(16 * q, 16)] = wq * vv

    for b in range(_NBUF):
        _issue(jnp.int32(b), b)

    def _outer(t):
        for b in range(_NBUF):
            k = _NBUF * t + b
            _wait_gathers(k, b)

            @pl.when(t > 0)
            def _():
                for cp in _scatter_cps(k - _NBUF, b):
                    cp.wait()

            _compute(b)
            for cp in _scatter_cps(k, b):
                cp.start(add=True)

            @pl.when(t < _CPW // _NBUF - 1)
            def _():
                _issue(k + _NBUF, b)
    _fori(_CPW // _NBUF, _outer)
    for b in range(_NBUF):
        for cp in _scatter_cps(jnp.int32(_CPW - _NBUF + b), b):
            cp.wait()

    plsc.subcore_barrier()
    _copy_out_stripe(c, s, acc, out_ref)
    _copy_out_stripe(c, s, accd, outd_ref)


_sc_l1 = pl.kernel(
    _sc_l1_body,
    out_type=[jax.ShapeDtypeStruct((_NC, _N, _D1), jnp.float32),
              jax.ShapeDtypeStruct((_NC, _N, 16), jnp.float32)],
    mesh=_mesh,
    scratch_types=[
        pltpu.VMEM((_CPW, _B), jnp.int32),
        pltpu.VMEM((_CPW, _B), jnp.int32),
        pltpu.VMEM((_NBUF, _B, 16), jnp.float32),
        pltpu.VMEM((_NBUF, _B, 16), jnp.float32),
        pltpu.VMEM((_NBUF, _B, _D1), jnp.bfloat16),
        pltpu.VMEM((_NBUF, _B, 16), jnp.float32),
        pltpu.VMEM((_NBUF, _B, _D1), jnp.float32),
        pltpu.VMEM_SHARED((_N + _GROW, _D1), jnp.float32),
        pltpu.VMEM_SHARED((_N + _GROW, 16), jnp.float32),
        pltpu.SemaphoreType.DMA,
        pltpu.SemaphoreType.DMA,
        pltpu.SemaphoreType.DMA,
        pltpu.SemaphoreType.DMA,
    ],
    compiler_params=pltpu.CompilerParams(needs_layout_passes=False,
                                         use_tc_tiling_on_sc=False),
)


def _sc_l2_body(src_ref, dst_ref, a2s_ref, a2d_ref, hwe_ref, out_ref,
                sidx2, didx2, a2sv, a2dv, hwg, wbuf, msg, acc, hwe_s,
                gsem0, gsem1, ssem0, ssem1):
    c = lax.axis_index("c")
    s = lax.axis_index("s")
    wid = jnp.int32(s) * _NC + jnp.int32(c)
    gsems = (gsem0, gsem1)
    ssems = (ssem0, ssem1)

    pltpu.sync_copy(src_ref.at[pl.ds(wid * _CPW, _CPW)], sidx2)
    pltpu.sync_copy(dst_ref.at[pl.ds(wid * _CPW, _CPW)], didx2)
    pltpu.sync_copy(a2s_ref, a2sv)
    pltpu.sync_copy(a2d_ref, a2dv)
    _stage_table(s, hwe_ref, hwe_s)
    _zero_acc_stripe(s, msg.at[jnp.int32(0)], acc)
    plsc.subcore_barrier()

    def _gather_cp(k, b):
        return pltpu.make_async_copy(hwe_s.at[sidx2.at[k]],
                                     hwg.at[jnp.int32(b)], gsems[b])

    def _scatter_cp(k, b):
        return pltpu.make_async_copy(msg.at[jnp.int32(b)], acc.at[didx2.at[k]], ssems[b])

    def _compute(k, b):
        bi = jnp.int32(b)
        hwgb, wbufb, msgb = hwg.at[bi], wbuf.at[bi], msg.at[bi]

        @plsc.parallel_loop(jnp.int32(0), jnp.int32(_B // 16), step=jnp.int32(1), unroll=2)
        def _grp(gi):
            sv = sidx2[k, pl.ds(16 * gi, 16)]
            dv = didx2[k, pl.ds(16 * gi, 16)]
            as16 = plsc.load_gather(a2sv, [sv])
            ad16 = plsc.load_gather(a2dv, [dv])
            lg = as16 + ad16
            lg = jnp.where(lg > 0, lg, 0.2 * lg)
            wbufb[pl.ds(16 * gi, 16)] = jnp.exp(lg)

        @plsc.parallel_loop(jnp.int32(0), jnp.int32(_B), step=jnp.int32(1), unroll=4)
        def _edge(e):
            esp = jnp.full((16,), e, jnp.int32)
            spl = plsc.load_gather(wbufb, [esp])
            for q in range(_D2 // 16):
                msgb[e, pl.ds(16 * q, 16)] = spl * hwgb[e, pl.ds(16 * q, 16)]

    for b in range(_NBUF):
        _gather_cp(jnp.int32(b), b).start()

    def _outer(t):
        for b in range(_NBUF):
            k = _NBUF * t + b
            _gather_cp(k, b).wait()

            @pl.when(t > 0)
            def _():
                _scatter_cp(k - _NBUF, b).wait()

            _compute(k, b)
            _scatter_cp(k, b).start(add=True)

            @pl.when(t < _CPW // _NBUF - 1)
            def _():
                _gather_cp(k + _NBUF, b).start()
    _fori(_CPW // _NBUF, _outer)
    for b in range(_NBUF):
        _scatter_cp(jnp.int32(_CPW - _NBUF + b), b).wait()

    plsc.subcore_barrier()
    _copy_out_stripe(c, s, acc, out_ref)


_sc_l2 = pl.kernel(
    _sc_l2_body,
    out_type=jax.ShapeDtypeStruct((_NC, _N, _D2), jnp.float32),
    mesh=_mesh,
    scratch_types=[
        pltpu.VMEM((_CPW, _B), jnp.int32),
        pltpu.VMEM((_CPW, _B), jnp.int32),
        pltpu.VMEM((_N,), jnp.float32),
        pltpu.VMEM((_N,), jnp.float32),
        pltpu.VMEM((_NBUF, _B, _D2), jnp.float32),
        pltpu.VMEM((_NBUF, _B), jnp.float32),
        pltpu.VMEM((_NBUF, _B, _D2), jnp.float32),
        pltpu.VMEM_SHARED((_N + _GROW, _D2), jnp.float32),
        pltpu.VMEM_SHARED((_N, _D2), jnp.float32),
        pltpu.SemaphoreType.DMA,
        pltpu.SemaphoreType.DMA,
        pltpu.SemaphoreType.DMA,
        pltpu.SemaphoreType.DMA,
    ],
    compiler_params=pltpu.CompilerParams(needs_layout_passes=False,
                                         use_tc_tiling_on_sc=False),
)


def _tc1_body(x_ref, w1_ref, as_ref, ad_ref, pm_ref, xwe_ref, asd_ref):
    xw = jnp.dot(x_ref[...], w1_ref[...], preferred_element_type=jnp.float32)
    a_s = jnp.dot(xw, as_ref[...], preferred_element_type=jnp.float32)
    a_d = jnp.dot(xw, ad_ref[...], preferred_element_type=jnp.float32)
    xwp = jnp.dot(xw, pm_ref[...], preferred_element_type=jnp.float32)
    xwe_ref[...] = xwp.astype(jnp.bfloat16)
    asd_ref[...] = jnp.concatenate([a_s, a_d], axis=1)


_tc1 = pl.pallas_call(
    _tc1_body,
    out_shape=[
        jax.ShapeDtypeStruct((_N, _D1), jnp.bfloat16),
        jax.ShapeDtypeStruct((_N, 16), jnp.float32),
    ],
)


def _tc2_body(acc_ref, accd_ref, b1_ref, w2_ref, a2s_ref, a2d_ref, r_ref,
              hwe_ref, a2_ref):
    num = acc_ref[0] + acc_ref[1]
    den8 = accd_ref[0][:, 0:8] + accd_ref[1][:, 0:8]
    den64 = jnp.dot(den8, r_ref[...], preferred_element_type=jnp.float32)
    h = num / jnp.maximum(den64, 1e-30) + b1_ref[...]
    h = jnp.where(h > 0, h, jnp.exp(jnp.minimum(h, 0.0)) - 1.0)
    hw = jnp.dot(h, w2_ref[...], preferred_element_type=jnp.float32)
    a2s = jnp.dot(hw, a2s_ref[...], preferred_element_type=jnp.float32)
    a2d = jnp.dot(hw, a2d_ref[...], preferred_element_type=jnp.float32)
    ones = jnp.ones((_N, 1), jnp.float32)
    zeros = jnp.zeros((_N, 7), jnp.float32)
    hwe_ref[...] = jnp.concatenate([hw, ones, zeros], axis=1)
    a2_ref[...] = jnp.concatenate([a2s, a2d], axis=1)


_tc2 = pl.pallas_call(
    _tc2_body,
    out_shape=[
        jax.ShapeDtypeStruct((_N, _D2), jnp.float32),
        jax.ShapeDtypeStruct((_N, 2), jnp.float32),
    ],
)


def _tc3_body(acc_ref, b2_ref, out_ref):
    num = acc_ref[0] + acc_ref[1]
    den = jnp.maximum(num[:, 40:41], 1e-30)
    lg = num[:, 0:40] / den + b2_ref[...]
    m = jnp.max(lg, axis=1, keepdims=True)
    ls = lg - m
    out_ref[...] = ls - jnp.log(jnp.sum(jnp.exp(ls), axis=1, keepdims=True))


_tc3 = pl.pallas_call(
    _tc3_body,
    out_shape=jax.ShapeDtypeStruct((_N, 40), jnp.float32),
)


def kernel(x, edge_index, y, W1, att_src1, att_dst1, b1, W2, att_src2,
           att_dst2, b2):
    del y
    f32 = jnp.float32
    x = x.astype(f32)
    src = edge_index[0].astype(jnp.int32)
    dst = edge_index[1].astype(jnp.int32)
    npad = _EPAD - _E
    src_p = jnp.pad(src, (0, npad)).reshape(_NW * _CPW, _B)
    dst_p = jnp.pad(dst, (0, npad),
                    constant_values=_N).reshape(_NW * _CPW, _B)

    hh = jnp.repeat(jnp.arange(8), 8)
    hmask = (hh[:, None] == jnp.arange(8)[None, :])
    As = jnp.where(hmask, att_src1.reshape(64).astype(f32)[:, None], 0.0)
    Ad = jnp.where(hmask, att_dst1.reshape(64).astype(f32)[:, None], 0.0)
    R = (jnp.arange(8)[:, None] == hh[None, :]).astype(f32)
    perm = []
    for half in range(2):
        for j in range(16):
            perm += [32 * half + j, 32 * half + 16 + j]
    Pm = (jnp.arange(64)[:, None] == jnp.array(perm)[None, :]).astype(f32)

    xwe, asd = _tc1(x, W1.astype(f32), As, Ad, Pm)
    acc1, acc1d = _sc_l1(src_p, dst_p, asd, xwe)
    hwe, a2 = _tc2(acc1, acc1d, b1.astype(f32).reshape(1, 64), W2.astype(f32),
                   att_src2.astype(f32).reshape(40, 1),
                   att_dst2.astype(f32).reshape(40, 1), R)
    acc2 = _sc_l2(src_p, dst_p, a2[:, 0], a2[:, 1], hwe)
    return _tc3(acc2, b2.astype(f32).reshape(1, 40))


# speedup vs baseline: 1.0049x; 1.0024x over previous
import jax
import jax.numpy as jnp
from jax import lax
from jax.experimental import pallas as pl
from jax.experimental.pallas import tpu as pltpu
from jax.experimental.pallas import tpu_sc as plsc

_N = 10000
_E = 320000
_NC, _NS = 2, 16
_NW = _NC * _NS
_B = 128
_CPW = 80
_NBUF = 2
_EPAD = _NW * _CPW * _B
_SPT = 632
_LAST = _N - (_NS - 1) * _SPT
_GROW = 128
_D1 = 64
_D2 = 48

_mesh = plsc.VectorSubcoreMesh(core_axis_name="c", subcore_axis_name="s")


def _iota16():
    return lax.iota(jnp.int32, 16)


def _fori(hi, body):
    return lax.fori_loop(jnp.int32(0), jnp.int32(hi),
                         lambda i, carry: (body(i), carry)[1], jnp.int32(0))


def _zero_acc_stripe(s, msg0, acc):
    d = msg0.shape[-1]
    z16 = jnp.zeros((16,), jnp.float32)

    def _zrow(e):
        for q in range(d // 16):
            msg0[e, pl.ds(16 * q, 16)] = z16
    _fori(8, _zrow)

    base_r = jnp.int32(s) * _SPT
    nz8 = jnp.where(s == _NS - 1, jnp.int32(_LAST // 8), jnp.int32(_SPT // 8))

    def _zacc(r):
        pltpu.sync_copy(msg0.at[pl.ds(0, 8)], acc.at[pl.ds(base_r + 8 * r, 8)])
    _fori(nz8, _zacc)


def _stage_table(s, hbm_ref, sp_ref):
    @pl.when(s < _NS - 1)
    def _():
        base = jnp.int32(s) * _SPT
        pltpu.sync_copy(hbm_ref.at[pl.ds(base, _SPT)],
                        sp_ref.at[pl.ds(base, _SPT)])

    @pl.when(s == _NS - 1)
    def _():
        pltpu.sync_copy(hbm_ref.at[pl.ds((_NS - 1) * _SPT, _LAST)],
                        sp_ref.at[pl.ds((_NS - 1) * _SPT, _LAST)])


def _copy_out_stripe(c, s, acc, out_ref):
    @pl.when(s < _NS - 1)
    def _():
        base = jnp.int32(s) * _SPT
        pltpu.sync_copy(acc.at[pl.ds(base, _SPT)],
                        out_ref.at[c, pl.ds(base, _SPT)])

    @pl.when(s == _NS - 1)
    def _():
        pltpu.sync_copy(acc.at[pl.ds((_NS - 1) * _SPT, _LAST)],
                        out_ref.at[c, pl.ds((_NS - 1) * _SPT, _LAST)])


def _sc_l1_body(src_ref, dst_ref, asd_ref, xwe_ref, out_ref, outd_ref,
                sidx2, didx2, asg, adg, xwg, wbuf, msg, acc, accd,
                gsem0, gsem1, ssem0, ssem1):
    c = lax.axis_index("c")
    s = lax.axis_index("s")
    wid = jnp.int32(s) * _NC + jnp.int32(c)
    it = _iota16()
    shift = (it & 7) + 8
    pats = [2 * q + (it >> 3) for q in range(4)]
    gsems = (gsem0, gsem1)
    ssems = (ssem0, ssem1)

    pltpu.sync_copy(src_ref.at[pl.ds(wid * _CPW, _CPW)], sidx2)
    pltpu.sync_copy(dst_ref.at[pl.ds(wid * _CPW, _CPW)], didx2)
    _zero_acc_stripe(s, msg.at[jnp.int32(0)], acc)
    _zero_acc_stripe(s, wbuf.at[jnp.int32(0)], accd)
    plsc.subcore_barrier()

    def _gather_cps(k, b):
        return (
            pltpu.make_async_copy(asd_ref.at[sidx2.at[k]], asg.at[jnp.int32(b)], gsems[b]),
            pltpu.make_async_copy(asd_ref.at[didx2.at[k]], adg.at[jnp.int32(b)], gsems[b]),
            pltpu.make_async_copy(xwe_ref.at[sidx2.at[k]], xwg.at[jnp.int32(b)], gsems[b]),
        )

    def _issue(k, b):
        for cp in _gather_cps(k, b):
            cp.start()

    def _wait_gathers(k, b):
        for cp in _gather_cps(k, b):
            cp.wait()

    def _scatter_cps(k, b):
        return (
            pltpu.make_async_copy(msg.at[jnp.int32(b)], acc.at[didx2.at[k]], ssems[b]),
            pltpu.make_async_copy(wbuf.at[jnp.int32(b)], accd.at[didx2.at[k]], ssems[b]),
        )

    def _compute(b):
        bi = jnp.int32(b)
        asgb, adgb, xwgb, wbufb, msgb = (
            asg.at[bi], adg.at[bi], xwg.at[bi], wbuf.at[bi], msg.at[bi])

        @plsc.parallel_loop(jnp.int32(0), jnp.int32(_B), step=jnp.int32(1), unroll=4)
        def _edge(e):
            esp = jnp.full((16,), e, jnp.int32)
            vas = asgb[e, :]
            vad = plsc.load_gather(adgb, [esp, shift])
            lg = vas + vad
            lg = jnp.where(lg > 0, lg, 0.2 * lg)
            w = jnp.exp(lg)
            wbufb[e, :] = w
            for q2 in range(_D1 // 32):
                v32 = xwgb[e, pl.ds(32 * q2, 32)]
                va, vb = plsc.unpack(v32, format=plsc.PackFormat.INTERLEAVED)
                for j, vv in enumerate((va, vb)):
                    q = 2 * q2 + j
                    wq = w[pats[q]]
                    msgb[e, pl.ds(16 * q, 16)] = wq * vv

    for b in range(_NBUF):
        _issue(jnp.int32(b), b)

    def _outer(t):
        for b in range(_NBUF):
            k = _NBUF * t + b
            _wait_gathers(k, b)

            @pl.when(t > 0)
            def _():
                for cp in _scatter_cps(k - _NBUF, b):
                    cp.wait()

            _compute(b)
            for cp in _scatter_cps(k, b):
                cp.start(add=True)

            @pl.when(t < _CPW // _NBUF - 1)
            def _():
                _issue(k + _NBUF, b)
    _fori(_CPW // _NBUF, _outer)
    for b in range(_NBUF):
        for cp in _scatter_cps(jnp.int32(_CPW - _NBUF + b), b):
            cp.wait()

    plsc.subcore_barrier()
    _copy_out_stripe(c, s, acc, out_ref)
    _copy_out_stripe(c, s, accd, outd_ref)


_sc_l1 = pl.kernel(
    _sc_l1_body,
    out_type=[jax.ShapeDtypeStruct((_NC, _N, _D1), jnp.float32),
              jax.ShapeDtypeStruct((_NC, _N, 16), jnp.float32)],
    mesh=_mesh,
    scratch_types=[
        pltpu.VMEM((_CPW, _B), jnp.int32),
        pltpu.VMEM((_CPW, _B), jnp.int32),
        pltpu.VMEM((_NBUF, _B, 16), jnp.float32),
        pltpu.VMEM((_NBUF, _B, 16), jnp.float32),
        pltpu.VMEM((_NBUF, _B, _D1), jnp.bfloat16),
        pltpu.VMEM((_NBUF, _B, 16), jnp.float32),
        pltpu.VMEM((_NBUF, _B, _D1), jnp.float32),
        pltpu.VMEM_SHARED((_N + _GROW, _D1), jnp.float32),
        pltpu.VMEM_SHARED((_N + _GROW, 16), jnp.float32),
        pltpu.SemaphoreType.DMA,
        pltpu.SemaphoreType.DMA,
        pltpu.SemaphoreType.DMA,
        pltpu.SemaphoreType.DMA,
    ],
    compiler_params=pltpu.CompilerParams(needs_layout_passes=False,
                                         use_tc_tiling_on_sc=False),
)


def _sc_l2_body(src_ref, dst_ref, a2s_ref, a2d_ref, hwe_ref, out_ref,
                sidx2, didx2, a2sv, a2dv, hwg, wbuf, msg, acc, hwe_s,
                gsem0, gsem1, ssem0, ssem1):
    c = lax.axis_index("c")
    s = lax.axis_index("s")
    wid = jnp.int32(s) * _NC + jnp.int32(c)
    gsems = (gsem0, gsem1)
    ssems = (ssem0, ssem1)

    pltpu.sync_copy(src_ref.at[pl.ds(wid * _CPW, _CPW)], sidx2)
    pltpu.sync_copy(dst_ref.at[pl.ds(wid * _CPW, _CPW)], didx2)
    pltpu.sync_copy(a2s_ref, a2sv)
    pltpu.sync_copy(a2d_ref, a2dv)
    _stage_table(s, hwe_ref, hwe_s)
    _zero_acc_stripe(s, msg.at[jnp.int32(0)], acc)
    plsc.subcore_barrier()

    def _gather_cp(k, b):
        return pltpu.make_async_copy(hwe_s.at[sidx2.at[k]],
                                     hwg.at[jnp.int32(b)], gsems[b])

    def _scatter_cp(k, b):
        return pltpu.make_async_copy(msg.at[jnp.int32(b)], acc.at[didx2.at[k]], ssems[b])

    def _compute(k, b):
        bi = jnp.int32(b)
        hwgb, wbufb, msgb = hwg.at[bi], wbuf.at[bi], msg.at[bi]

        @plsc.parallel_loop(jnp.int32(0), jnp.int32(_B // 16), step=jnp.int32(1), unroll=2)
        def _grp(gi):
            sv = sidx2[k, pl.ds(16 * gi, 16)]
            dv = didx2[k, pl.ds(16 * gi, 16)]
            as16 = plsc.load_gather(a2sv, [sv])
            ad16 = plsc.load_gather(a2dv, [dv])
            lg = as16 + ad16
            lg = jnp.where(lg > 0, lg, 0.2 * lg)
            wbufb[pl.ds(16 * gi, 16)] = jnp.exp(lg)

        @plsc.parallel_loop(jnp.int32(0), jnp.int32(_B), step=jnp.int32(1), unroll=8)
        def _edge(e):
            esp = jnp.full((16,), e, jnp.int32)
            spl = plsc.load_gather(wbufb, [esp])
            for q in range(_D2 // 16):
                msgb[e, pl.ds(16 * q, 16)] = spl * hwgb[e, pl.ds(16 * q, 16)]

    for b in range(_NBUF):
        _gather_cp(jnp.int32(b), b).start()

    def _outer(t):
        for b in range(_NBUF):
            k = _NBUF * t + b
            _gather_cp(k, b).wait()

            @pl.when(t > 0)
            def _():
                _scatter_cp(k - _NBUF, b).wait()

            _compute(k, b)
            _scatter_cp(k, b).start(add=True)

            @pl.when(t < _CPW // _NBUF - 1)
            def _():
                _gather_cp(k + _NBUF, b).start()
    _fori(_CPW // _NBUF, _outer)
    for b in range(_NBUF):
        _scatter_cp(jnp.int32(_CPW - _NBUF + b), b).wait()

    plsc.subcore_barrier()
    _copy_out_stripe(c, s, acc, out_ref)


_sc_l2 = pl.kernel(
    _sc_l2_body,
    out_type=jax.ShapeDtypeStruct((_NC, _N, _D2), jnp.float32),
    mesh=_mesh,
    scratch_types=[
        pltpu.VMEM((_CPW, _B), jnp.int32),
        pltpu.VMEM((_CPW, _B), jnp.int32),
        pltpu.VMEM((_N,), jnp.float32),
        pltpu.VMEM((_N,), jnp.float32),
        pltpu.VMEM((_NBUF, _B, _D2), jnp.float32),
        pltpu.VMEM((_NBUF, _B), jnp.float32),
        pltpu.VMEM((_NBUF, _B, _D2), jnp.float32),
        pltpu.VMEM_SHARED((_N + _GROW, _D2), jnp.float32),
        pltpu.VMEM_SHARED((_N, _D2), jnp.float32),
        pltpu.SemaphoreType.DMA,
        pltpu.SemaphoreType.DMA,
        pltpu.SemaphoreType.DMA,
        pltpu.SemaphoreType.DMA,
    ],
    compiler_params=pltpu.CompilerParams(needs_layout_passes=False,
                                         use_tc_tiling_on_sc=False),
)


def _tc1_body(x_ref, w1_ref, as_ref, ad_ref, pm_ref, xwe_ref, asd_ref):
    xw = jnp.dot(x_ref[...], w1_ref[...], preferred_element_type=jnp.float32)
    a_s = jnp.dot(xw, as_ref[...], preferred_element_type=jnp.float32)
    a_d = jnp.dot(xw, ad_ref[...], preferred_element_type=jnp.float32)
    xwp = jnp.dot(xw, pm_ref[...], preferred_element_type=jnp.float32)
    xwe_ref[...] = xwp.astype(jnp.bfloat16)
    asd_ref[...] = jnp.concatenate([a_s, a_d], axis=1)


_tc1 = pl.pallas_call(
    _tc1_body,
    out_shape=[
        jax.ShapeDtypeStruct((_N, _D1), jnp.bfloat16),
        jax.ShapeDtypeStruct((_N, 16), jnp.float32),
    ],
)


def _tc2_body(acc_ref, accd_ref, b1_ref, w2_ref, a2s_ref, a2d_ref, r_ref,
              hwe_ref, a2_ref):
    num = acc_ref[0] + acc_ref[1]
    den8 = accd_ref[0][:, 0:8] + accd_ref[1][:, 0:8]
    den64 = jnp.dot(den8, r_ref[...], preferred_element_type=jnp.float32)
    h = num / jnp.maximum(den64, 1e-30) + b1_ref[...]
    h = jnp.where(h > 0, h, jnp.exp(jnp.minimum(h, 0.0)) - 1.0)
    hw = jnp.dot(h, w2_ref[...], preferred_element_type=jnp.float32)
    a2s = jnp.dot(hw, a2s_ref[...], preferred_element_type=jnp.float32)
    a2d = jnp.dot(hw, a2d_ref[...], preferred_element_type=jnp.float32)
    ones = jnp.ones((_N, 1), jnp.float32)
    zeros = jnp.zeros((_N, 7), jnp.float32)
    hwe_ref[...] = jnp.concatenate([hw, ones, zeros], axis=1)
    a2_ref[...] = jnp.concatenate([a2s, a2d], axis=1)


_tc2 = pl.pallas_call(
    _tc2_body,
    out_shape=[
        jax.ShapeDtypeStruct((_N, _D2), jnp.float32),
        jax.ShapeDtypeStruct((_N, 2), jnp.float32),
    ],
)


def _tc3_body(acc_ref, b2_ref, out_ref):
    num = acc_ref[0] + acc_ref[1]
    den = jnp.maximum(num[:, 40:41], 1e-30)
    lg = num[:, 0:40] / den + b2_ref[...]
    m = jnp.max(lg, axis=1, keepdims=True)
    ls = lg - m
    out_ref[...] = ls - jnp.log(jnp.sum(jnp.exp(ls), axis=1, keepdims=True))


_tc3 = pl.pallas_call(
    _tc3_body,
    out_shape=jax.ShapeDtypeStruct((_N, 40), jnp.float32),
)


def kernel(x, edge_index, y, W1, att_src1, att_dst1, b1, W2, att_src2,
           att_dst2, b2):
    del y
    f32 = jnp.float32
    x = x.astype(f32)
    src = edge_index[0].astype(jnp.int32)
    dst = edge_index[1].astype(jnp.int32)
    npad = _EPAD - _E
    src_p = jnp.pad(src, (0, npad)).reshape(_NW * _CPW, _B)
    dst_p = jnp.pad(dst, (0, npad),
                    constant_values=_N).reshape(_NW * _CPW, _B)

    hh = jnp.repeat(jnp.arange(8), 8)
    hmask = (hh[:, None] == jnp.arange(8)[None, :])
    As = jnp.where(hmask, att_src1.reshape(64).astype(f32)[:, None], 0.0)
    Ad = jnp.where(hmask, att_dst1.reshape(64).astype(f32)[:, None], 0.0)
    R = (jnp.arange(8)[:, None] == hh[None, :]).astype(f32)
    perm = []
    for half in range(2):
        for j in range(16):
            perm += [32 * half + j, 32 * half + 16 + j]
    Pm = (jnp.arange(64)[:, None] == jnp.array(perm)[None, :]).astype(f32)

    xwe, asd = _tc1(x, W1.astype(f32), As, Ad, Pm)
    acc1, acc1d = _sc_l1(src_p, dst_p, asd, xwe)
    hwe, a2 = _tc2(acc1, acc1d, b1.astype(f32).reshape(1, 64), W2.astype(f32),
                   att_src2.astype(f32).reshape(40, 1),
                   att_dst2.astype(f32).reshape(40, 1), R)
    acc2 = _sc_l2(src_p, dst_p, a2[:, 0], a2[:, 1], hwe)
    return _tc3(acc2, b2.astype(f32).reshape(1, 40))
